# Initial kernel scaffold; baseline (speedup 1.0000x reference)
import jax
import jax.numpy as jnp
from jax.experimental import pallas as pl


def kernel(x, edge_index, edge_attr, W1, b1, W2, b2):
    raise NotImplementedError("write your pallas kernel here")


# speedup vs baseline: 3.9292x; 3.9292x over previous
import functools

import jax
import jax.numpy as jnp
from jax import lax
from jax.experimental import pallas as pl
from jax.experimental.pallas import tpu as pltpu
from jax.experimental.pallas import tpu_sc as plsc

N_NODES = 10000
N_EDGES = 320000
D = 128
NC, NS = 2, 16
NW = NC * NS
C = 128
N_CHUNKS = N_EDGES // C
FULL = N_CHUNKS // NW
REM = N_CHUNKS - FULL * NW
RPT = 624
TAIL_ROWS = N_NODES - RPT * NS


def _sc_aggregate(x, src, dst, edge_attr):
    mesh = plsc.VectorSubcoreMesh(core_axis_name="c", subcore_axis_name="s")

    @functools.partial(
        pl.kernel,
        out_type=jax.ShapeDtypeStruct((NC, N_NODES, D), jnp.float32),
        mesh=mesh,
        scratch_types=[
            pltpu.VMEM_SHARED((N_NODES, D), jnp.float32),
            pltpu.VMEM((C, D), jnp.float32),
            pltpu.VMEM((C, D), jnp.float32),
            pltpu.VMEM((C,), jnp.int32),
            pltpu.VMEM((C,), jnp.int32),
            pltpu.SemaphoreType.DMA,
        ],
    )
    def k(x_hbm, src_hbm, dst_hbm, ea_hbm, out_hbm,
          aggr_s, msg_v, xg_v, sidx_v, didx_v, sem):
        cid = lax.axis_index("c")
        sid = lax.axis_index("s")
        wid = sid * NC + cid

        zero16 = jnp.zeros((16,), jnp.float32)

        def zrow(r, carry):
            for j in range(D // 16):
                xg_v[r, pl.ds(j * 16, 16)] = zero16
            return carry

        lax.fori_loop(0, C, zrow, 0)
        row0 = sid * RPT
        for q in range(4):
            pltpu.sync_copy(xg_v, aggr_s.at[pl.ds(row0 + q * C, C)])
        tail = RPT - 4 * C
        pltpu.sync_copy(xg_v.at[pl.ds(0, tail)],
                        aggr_s.at[pl.ds(row0 + 4 * C, tail)])

        @pl.when(sid == NS - 1)
        def _():
            pltpu.sync_copy(xg_v.at[pl.ds(0, TAIL_ROWS)],
                            aggr_s.at[pl.ds(NS * RPT, TAIL_ROWS)])

        plsc.subcore_barrier()

        def chunk_body(c, carry):
            base = (c * NW + wid) * C
            pltpu.sync_copy(src_hbm.at[pl.ds(base, C)], sidx_v)
            pltpu.sync_copy(dst_hbm.at[pl.ds(base, C)], didx_v)
            pltpu.sync_copy(ea_hbm.at[pl.ds(base, C)], msg_v)
            pltpu.async_copy(x_hbm.at[sidx_v], xg_v, sem).wait()

            def rbody(r, rc):
                for j in range(D // 16):
                    sl = pl.ds(j * 16, 16)
                    msg_v[r, sl] = jnp.maximum(msg_v[r, sl] + xg_v[r, sl], 0.0)
                return rc

            lax.fori_loop(0, C, rbody, 0)
            pltpu.sync_copy(msg_v, aggr_s.at[didx_v], add=True)
            return carry

        lax.fori_loop(0, FULL, chunk_body, 0)

        @pl.when(wid < REM)
        def _():
            chunk_body(FULL, 0)

        plsc.subcore_barrier()
        pltpu.sync_copy(aggr_s.at[pl.ds(row0, RPT)],
                        out_hbm.at[cid, pl.ds(row0, RPT)])

        @pl.when(sid == NS - 1)
        def _():
            pltpu.sync_copy(aggr_s.at[pl.ds(NS * RPT, TAIL_ROWS)],
                            out_hbm.at[cid, pl.ds(NS * RPT, TAIL_ROWS)])

    return k(x, src, dst, edge_attr)


def _tc_mlp(x, a0, a1, W1, b1, W2, b2):
    R = 400

    def body(x_ref, a0_ref, a1_ref, w1_ref, b1_ref, w2_ref, b2_ref, o_ref):
        xb = x_ref[...]
        h = xb + a0_ref[...] + a1_ref[...]
        h = jnp.maximum(
            jnp.dot(h, w1_ref[...], preferred_element_type=jnp.float32)
            + b1_ref[...], 0.0)
        h = jnp.maximum(
            jnp.dot(h, w2_ref[...], preferred_element_type=jnp.float32)
            + b2_ref[...], 0.0)
        o_ref[...] = xb + h

    return pl.pallas_call(
        body,
        grid=(N_NODES // R,),
        in_specs=[
            pl.BlockSpec((R, D), lambda i: (i, 0)),
            pl.BlockSpec((R, D), lambda i: (i, 0)),
            pl.BlockSpec((R, D), lambda i: (i, 0)),
            pl.BlockSpec((D, D), lambda i: (0, 0)),
            pl.BlockSpec((1, D), lambda i: (0, 0)),
            pl.BlockSpec((D, D), lambda i: (0, 0)),
            pl.BlockSpec((1, D), lambda i: (0, 0)),
        ],
        out_specs=pl.BlockSpec((R, D), lambda i: (i, 0)),
        out_shape=jax.ShapeDtypeStruct((N_NODES, D), jnp.float32),
    )(x, a0, a1, W1, b1.reshape(1, D), W2, b2.reshape(1, D))


def kernel(x, edge_index, edge_attr, W1, b1, W2, b2):
    ei = edge_index.astype(jnp.int32)
    aggr = _sc_aggregate(x, ei[0], ei[1], edge_attr)
    return _tc_mlp(x, aggr[0], aggr[1], W1, b1, W2, b2)
